# baseline (speedup 1.0000x reference)
import jax
import jax.numpy as jnp
import numpy as np
from jax.experimental import pallas as pl

B = 1
N = 2048
D_MODEL = 1024
NUM_HEADS = 16
HEAD_DIM = D_MODEL // NUM_HEADS
QBLK = 256

_QSCALE = 0.125 * 1.4426950408889634


def _qkv_kernel(xq_ref, xk_ref, xv_ref, wq_ref, wk_ref, wv_ref,
                bq_ref, bk_ref, bv_ref, q_ref, k_ref, v_ref):
    dims = (((0,), (1,)), ((), ()))
    xq = xq_ref[...].astype(jnp.bfloat16)
    xk = xk_ref[...].astype(jnp.bfloat16)
    xv = xv_ref[...].astype(jnp.bfloat16)
    q = jax.lax.dot_general(xq, wq_ref[...], dims,
                            preferred_element_type=jnp.float32) + bq_ref[...]
    q_ref[...] = (q * _QSCALE).astype(jnp.bfloat16)
    k = jax.lax.dot_general(xk, wk_ref[...], dims,
                            preferred_element_type=jnp.float32) + bk_ref[...]
    k_ref[...] = k.astype(jnp.bfloat16)
    v = jax.lax.dot_general(xv, wv_ref[...], dims,
                            preferred_element_type=jnp.float32) + bv_ref[...]
    v_ref[...] = v.astype(jnp.bfloat16)


def _attn_kernel(qlab_ref, vlab_ref, q_ref, k_ref, v_ref, wm_ref, bm_ref,
                 ones_ref, o_ref):
    mask = qlab_ref[...] == vlab_ref[...]
    neg = jnp.where(mask, 0.0, -1e30)
    has = jnp.any(mask, axis=-1, keepdims=True).astype(jnp.float32)
    outs = []
    for h in range(NUM_HEADS):
        sl = slice(h * HEAD_DIM, (h + 1) * HEAD_DIM)
        q = q_ref[:, sl]
        k = k_ref[:, sl]
        s = jax.lax.dot_general(q, k, (((1,), (1,)), ((), ())),
                                preferred_element_type=jnp.float32)
        masked = s + neg
        m = jnp.max(masked, axis=-1, keepdims=True)
        e = jnp.exp2(masked - m).astype(jnp.bfloat16)
        denom = jnp.dot(e, ones_ref[...], preferred_element_type=jnp.float32)
        o = jnp.dot(e, v_ref[:, sl], preferred_element_type=jnp.float32)
        outs.append((o * (has / denom)).astype(jnp.bfloat16))
    o_all = jnp.concatenate(outs, axis=1)
    o_ref[...] = jax.lax.dot_general(
        wm_ref[...], o_all, (((1,), (1,)), ((), ())),
        preferred_element_type=jnp.float32) + bm_ref[...]


def _rowperm(W):
    return W.reshape(HEAD_DIM, NUM_HEADS, D_MODEL).transpose(1, 0, 2) \
            .reshape(D_MODEL, D_MODEL)


def _rowperm_b(b):
    return b.reshape(HEAD_DIM, NUM_HEADS).T.reshape(1, D_MODEL)


@jax.jit
def kernel(query, key, value, query_labels, value_labels,
           Wq, bq, Wk, bk, Wv, bv, Wm, bm):
    xq, xk, xv = query[0], key[0], value[0]
    WqR, WkR, WvR = (_rowperm(Wq).astype(jnp.bfloat16),
                     _rowperm(Wk).astype(jnp.bfloat16),
                     _rowperm(Wv).astype(jnp.bfloat16))
    bqR, bkR, bvR = _rowperm_b(bq), _rowperm_b(bk), _rowperm_b(bv)
    WmP = Wm.reshape(D_MODEL, HEAD_DIM, NUM_HEADS).transpose(0, 2, 1) \
            .reshape(D_MODEL, D_MODEL).astype(jnp.bfloat16)
    bmR = bm.reshape(D_MODEL, 1)

    nblk = N // QBLK
    q2, k2, v2 = pl.pallas_call(
        _qkv_kernel,
        grid=(nblk,),
        in_specs=[
            pl.BlockSpec((D_MODEL, QBLK), lambda i: (0, i)),
            pl.BlockSpec((D_MODEL, QBLK), lambda i: (0, i)),
            pl.BlockSpec((D_MODEL, QBLK), lambda i: (0, i)),
            pl.BlockSpec((D_MODEL, D_MODEL), lambda i: (0, 0)),
            pl.BlockSpec((D_MODEL, D_MODEL), lambda i: (0, 0)),
            pl.BlockSpec((D_MODEL, D_MODEL), lambda i: (0, 0)),
            pl.BlockSpec((1, D_MODEL), lambda i: (0, 0)),
            pl.BlockSpec((1, D_MODEL), lambda i: (0, 0)),
            pl.BlockSpec((1, D_MODEL), lambda i: (0, 0)),
        ],
        out_specs=[
            pl.BlockSpec((QBLK, D_MODEL), lambda i: (i, 0)),
            pl.BlockSpec((QBLK, D_MODEL), lambda i: (i, 0)),
            pl.BlockSpec((QBLK, D_MODEL), lambda i: (i, 0)),
        ],
        out_shape=[jax.ShapeDtypeStruct((N, D_MODEL), jnp.bfloat16)] * 3,
    )(xq, xk, xv, WqR, WkR, WvR, bqR, bkR, bvR)

    qlab = query_labels[0].reshape(N, 1)
    vlab = value_labels[0].reshape(1, N)
    ones_col = jnp.ones((N, 1), dtype=jnp.bfloat16)
    out = pl.pallas_call(
        _attn_kernel,
        grid=(nblk,),
        in_specs=[
            pl.BlockSpec((QBLK, 1), lambda i: (i, 0)),
            pl.BlockSpec((1, N), lambda i: (0, 0)),
            pl.BlockSpec((QBLK, D_MODEL), lambda i: (i, 0)),
            pl.BlockSpec((N, D_MODEL), lambda i: (0, 0)),
            pl.BlockSpec((N, D_MODEL), lambda i: (0, 0)),
            pl.BlockSpec((D_MODEL, D_MODEL), lambda i: (0, 0)),
            pl.BlockSpec((D_MODEL, 1), lambda i: (0, 0)),
            pl.BlockSpec((N, 1), lambda i: (0, 0)),
        ],
        out_specs=pl.BlockSpec((D_MODEL, QBLK), lambda i: (0, i)),
        out_shape=jax.ShapeDtypeStruct((D_MODEL, N), jnp.float32),
    )(qlab, vlab, q2, k2, v2, WmP, bmR, ones_col)

    return out[None]


# speedup vs baseline: 1.2096x; 1.2096x over previous
import jax
import jax.numpy as jnp
import numpy as np
from jax.experimental import pallas as pl

B = 1
N = 2048
D_MODEL = 1024
NUM_HEADS = 16
HEAD_DIM = D_MODEL // NUM_HEADS
QBLK = 256
AUG = 128
N_CLUSTERS = 16

_QSCALE = 0.125 * 1.4426950408889634
_CBIG = 100.0


def _head_norms(x):
    cols = []
    for h in range(NUM_HEADS):
        sl = slice(h * HEAD_DIM, (h + 1) * HEAD_DIM)
        cols.append(jnp.sum(x[:, sl] * x[:, sl], axis=1, keepdims=True))
    return jnp.sqrt(jnp.concatenate(cols, axis=1))


def _qkv_kernel(xq_ref, xk_ref, xv_ref, vlab_ref, wq_ref, wk_ref, wv_ref,
                bq_ref, bk_ref, bv_ref,
                q_ref, qn_ref, k_ref, kn_ref, v_ref):
    dims = (((0,), (1,)), ((), ()))
    xq = xq_ref[...].astype(jnp.bfloat16)
    xk = xk_ref[...].astype(jnp.bfloat16)
    xv = xv_ref[...].astype(jnp.bfloat16)

    q = jax.lax.dot_general(xq, wq_ref[...], dims,
                            preferred_element_type=jnp.float32) + bq_ref[...]
    q = q * _QSCALE
    q_ref[...] = q.astype(jnp.bfloat16)
    qn_ref[...] = _head_norms(q)

    k = jax.lax.dot_general(xk, wk_ref[...], dims,
                            preferred_element_type=jnp.float32) + bk_ref[...]
    kn_ref[...] = _head_norms(k)
    kb = k.astype(jnp.bfloat16)

    v = jax.lax.dot_general(xv, wv_ref[...], dims,
                            preferred_element_type=jnp.float32) + bv_ref[...]
    vb = v.astype(jnp.bfloat16)

    vlab = vlab_ref[...]
    cl = jax.lax.broadcasted_iota(jnp.int32, (1, N_CLUSTERS), 1)
    Lv = (vlab == cl).astype(jnp.bfloat16)
    ones = jnp.ones((QBLK, 1), dtype=jnp.bfloat16)
    zk = jnp.zeros((QBLK, AUG - HEAD_DIM - 1 - N_CLUSTERS), jnp.bfloat16)
    zv = jnp.zeros((QBLK, AUG - HEAD_DIM - 1), jnp.bfloat16)

    kp, vp = [], []
    for h in range(NUM_HEADS):
        sl = slice(h * HEAD_DIM, (h + 1) * HEAD_DIM)
        kp += [kb[:, sl], ones, Lv, zk]
        vp += [vb[:, sl], ones, zv]
    k_ref[...] = jnp.concatenate(kp, axis=1)
    v_ref[...] = jnp.concatenate(vp, axis=1)


def _attn_kernel(qlab_ref, vlab_ref, q_ref, qn_ref, k_ref, kn_ref, v_ref,
                 wm_ref, bm_ref, o_ref):
    maxk = jnp.max(kn_ref[...], axis=0, keepdims=True)
    M = qn_ref[...] * maxk * 1.01 + 0.5
    negMC = (-(M + _CBIG)).astype(jnp.bfloat16)

    qlab = qlab_ref[...]
    cl = jax.lax.broadcasted_iota(jnp.int32, (1, N_CLUSTERS), 1)
    Lq = qlab == cl
    LqC = jnp.where(Lq, _CBIG, 0.0).astype(jnp.bfloat16)

    vlab = vlab_ref[...]
    cl2 = jax.lax.broadcasted_iota(jnp.int32, (N_CLUSTERS, 1), 0)
    cnt = jnp.sum((vlab == cl2).astype(jnp.float32), axis=1, keepdims=True)
    own = jnp.sum(jnp.where(Lq, cnt.T, 0.0), axis=1, keepdims=True)
    has = (own > 0.0).astype(jnp.float32)

    q2 = q_ref[...]
    zq = jnp.zeros((QBLK, AUG - HEAD_DIM - 1 - N_CLUSTERS), jnp.bfloat16)
    outs = []
    for h in range(NUM_HEADS):
        sl = slice(h * HEAD_DIM, (h + 1) * HEAD_DIM)
        sa = slice(h * AUG, (h + 1) * AUG)
        q_aug = jnp.concatenate(
            [q2[:, sl], negMC[:, h:h + 1], LqC, zq], axis=1)
        s = jax.lax.dot_general(q_aug, k_ref[:, sa], (((1,), (1,)), ((), ())),
                                preferred_element_type=jnp.float32)
        e = jnp.exp2(s).astype(jnp.bfloat16)
        o_aug = jnp.dot(e, v_ref[:, sa], preferred_element_type=jnp.float32)
        o = o_aug[:, :HEAD_DIM]
        denom = o_aug[:, HEAD_DIM:HEAD_DIM + 1]
        outs.append((o * (has / jnp.maximum(denom, 1e-30))).astype(jnp.bfloat16))
    o_all = jnp.concatenate(outs, axis=1)
    o_ref[...] = jax.lax.dot_general(
        wm_ref[...], o_all, (((1,), (1,)), ((), ())),
        preferred_element_type=jnp.float32) + bm_ref[...]


def _rowperm(W):
    return W.reshape(HEAD_DIM, NUM_HEADS, D_MODEL).transpose(1, 0, 2) \
            .reshape(D_MODEL, D_MODEL)


def _rowperm_b(b):
    return b.reshape(HEAD_DIM, NUM_HEADS).T.reshape(1, D_MODEL)


@jax.jit
def kernel(query, key, value, query_labels, value_labels,
           Wq, bq, Wk, bk, Wv, bv, Wm, bm):
    xq, xk, xv = query[0], key[0], value[0]
    WqR, WkR, WvR = (_rowperm(Wq).astype(jnp.bfloat16),
                     _rowperm(Wk).astype(jnp.bfloat16),
                     _rowperm(Wv).astype(jnp.bfloat16))
    bqR, bkR, bvR = _rowperm_b(bq), _rowperm_b(bk), _rowperm_b(bv)
    WmP = Wm.reshape(D_MODEL, HEAD_DIM, NUM_HEADS).transpose(0, 2, 1) \
            .reshape(D_MODEL, D_MODEL).astype(jnp.bfloat16)
    bmR = bm.reshape(D_MODEL, 1)

    qlab = query_labels[0].reshape(N, 1)
    vlab_col = value_labels[0].reshape(N, 1)
    vlab_row = value_labels[0].reshape(1, N)
    NAUG = NUM_HEADS * AUG

    nblk = N // QBLK
    q2, qn, k_aug, kn, v_aug = pl.pallas_call(
        _qkv_kernel,
        grid=(nblk,),
        in_specs=[
            pl.BlockSpec((D_MODEL, QBLK), lambda i: (0, i)),
            pl.BlockSpec((D_MODEL, QBLK), lambda i: (0, i)),
            pl.BlockSpec((D_MODEL, QBLK), lambda i: (0, i)),
            pl.BlockSpec((QBLK, 1), lambda i: (i, 0)),
            pl.BlockSpec((D_MODEL, D_MODEL), lambda i: (0, 0)),
            pl.BlockSpec((D_MODEL, D_MODEL), lambda i: (0, 0)),
            pl.BlockSpec((D_MODEL, D_MODEL), lambda i: (0, 0)),
            pl.BlockSpec((1, D_MODEL), lambda i: (0, 0)),
            pl.BlockSpec((1, D_MODEL), lambda i: (0, 0)),
            pl.BlockSpec((1, D_MODEL), lambda i: (0, 0)),
        ],
        out_specs=[
            pl.BlockSpec((QBLK, D_MODEL), lambda i: (i, 0)),
            pl.BlockSpec((QBLK, NUM_HEADS), lambda i: (i, 0)),
            pl.BlockSpec((QBLK, NAUG), lambda i: (i, 0)),
            pl.BlockSpec((QBLK, NUM_HEADS), lambda i: (i, 0)),
            pl.BlockSpec((QBLK, NAUG), lambda i: (i, 0)),
        ],
        out_shape=[
            jax.ShapeDtypeStruct((N, D_MODEL), jnp.bfloat16),
            jax.ShapeDtypeStruct((N, NUM_HEADS), jnp.float32),
            jax.ShapeDtypeStruct((N, NAUG), jnp.bfloat16),
            jax.ShapeDtypeStruct((N, NUM_HEADS), jnp.float32),
            jax.ShapeDtypeStruct((N, NAUG), jnp.bfloat16),
        ],
    )(xq, xk, xv, vlab_col, WqR, WkR, WvR, bqR, bkR, bvR)

    out = pl.pallas_call(
        _attn_kernel,
        grid=(nblk,),
        in_specs=[
            pl.BlockSpec((QBLK, 1), lambda i: (i, 0)),
            pl.BlockSpec((1, N), lambda i: (0, 0)),
            pl.BlockSpec((QBLK, D_MODEL), lambda i: (i, 0)),
            pl.BlockSpec((QBLK, NUM_HEADS), lambda i: (i, 0)),
            pl.BlockSpec((N, NAUG), lambda i: (0, 0)),
            pl.BlockSpec((N, NUM_HEADS), lambda i: (0, 0)),
            pl.BlockSpec((N, NAUG), lambda i: (0, 0)),
            pl.BlockSpec((D_MODEL, D_MODEL), lambda i: (0, 0)),
            pl.BlockSpec((D_MODEL, 1), lambda i: (0, 0)),
        ],
        out_specs=pl.BlockSpec((D_MODEL, QBLK), lambda i: (0, i)),
        out_shape=jax.ShapeDtypeStruct((D_MODEL, N), jnp.float32),
    )(qlab, vlab_row, q2, qn, k_aug, kn, v_aug, WmP, bmR)

    return out[None]


# speedup vs baseline: 1.3500x; 1.1161x over previous
import jax
import jax.numpy as jnp
import numpy as np
from jax.experimental import pallas as pl

B = 1
N = 2048
D_MODEL = 1024
NUM_HEADS = 16
HEAD_DIM = D_MODEL // NUM_HEADS
QBLK = 512
AUG = 128
N_CLUSTERS = 16

_QSCALE = 0.125 * 1.4426950408889634
_CBIG = 100.0


def _head_norms(x):
    cols = []
    for h in range(NUM_HEADS):
        sl = slice(h * HEAD_DIM, (h + 1) * HEAD_DIM)
        cols.append(jnp.sum(x[:, sl] * x[:, sl], axis=1, keepdims=True))
    return jnp.sqrt(jnp.concatenate(cols, axis=1))


def _qkv_kernel(xq_ref, xk_ref, xv_ref, wq_ref, wk_ref, wv_ref,
                bq_ref, bk_ref, bv_ref,
                q_ref, qn_ref, k_ref, kn_ref, v_ref):
    dims = (((0,), (1,)), ((), ()))
    xq = xq_ref[...].astype(jnp.bfloat16)
    xk = xk_ref[...].astype(jnp.bfloat16)
    xv = xv_ref[...].astype(jnp.bfloat16)

    q = jax.lax.dot_general(xq, wq_ref[...], dims,
                            preferred_element_type=jnp.float32) + bq_ref[...]
    q = q * _QSCALE
    q_ref[...] = q.astype(jnp.bfloat16)
    qn_ref[...] = _head_norms(q)

    k = jax.lax.dot_general(xk, wk_ref[...], dims,
                            preferred_element_type=jnp.float32) + bk_ref[...]
    kn_ref[...] = _head_norms(k)
    kb = k.astype(jnp.bfloat16)

    v = jax.lax.dot_general(xv, wv_ref[...], dims,
                            preferred_element_type=jnp.float32) + bv_ref[...]
    vb = v.astype(jnp.bfloat16)

    k_ref[...] = kb
    ones = jnp.ones((QBLK, 1), dtype=jnp.bfloat16)
    zv = jnp.zeros((QBLK, AUG - HEAD_DIM - 1), jnp.bfloat16)
    vp = []
    for h in range(NUM_HEADS):
        sl = slice(h * HEAD_DIM, (h + 1) * HEAD_DIM)
        vp += [vb[:, sl], ones, zv]
    v_ref[...] = jnp.concatenate(vp, axis=1)


def _attn_kernel(qlab_ref, vlab_ref, q_ref, qn_ref, k_ref, kn_ref, v_ref,
                 wm_ref, bm_ref, o_ref):
    maxk = jnp.max(kn_ref[...], axis=0, keepdims=True)
    M = jnp.max(qn_ref[...] * maxk, axis=1, keepdims=True) * 1.01 + 0.5

    qlab = qlab_ref[...]
    vlab = vlab_ref[...]
    match = qlab == vlab
    aug = jnp.where(match, -M, -M - _CBIG)

    cl = jax.lax.broadcasted_iota(jnp.int32, (1, N_CLUSTERS), 1)
    Lq = qlab == cl
    cl2 = jax.lax.broadcasted_iota(jnp.int32, (N_CLUSTERS, 1), 0)
    cnt = jnp.sum((vlab == cl2).astype(jnp.float32), axis=1, keepdims=True)
    own = jnp.sum(jnp.where(Lq, cnt.T, 0.0), axis=1, keepdims=True)
    has = (own > 0.0).astype(jnp.float32)

    q2 = q_ref[...]
    outs = []
    for h in range(NUM_HEADS):
        sl = slice(h * HEAD_DIM, (h + 1) * HEAD_DIM)
        sa = slice(h * AUG, (h + 1) * AUG)
        s = jax.lax.dot_general(q2[:, sl], k_ref[:, sl],
                                (((1,), (1,)), ((), ())),
                                preferred_element_type=jnp.float32) + aug
        e = jnp.exp2(s).astype(jnp.bfloat16)
        o_aug = jnp.dot(e, v_ref[:, sa], preferred_element_type=jnp.float32)
        o = o_aug[:, :HEAD_DIM]
        denom = o_aug[:, HEAD_DIM:HEAD_DIM + 1]
        outs.append((o * (has / jnp.maximum(denom, 1e-30))).astype(jnp.bfloat16))
    o_all = jnp.concatenate(outs, axis=1)
    o_ref[...] = jax.lax.dot_general(
        wm_ref[...], o_all, (((1,), (1,)), ((), ())),
        preferred_element_type=jnp.float32) + bm_ref[...]


def _rowperm(W):
    return W.reshape(HEAD_DIM, NUM_HEADS, D_MODEL).transpose(1, 0, 2) \
            .reshape(D_MODEL, D_MODEL)


def _rowperm_b(b):
    return b.reshape(HEAD_DIM, NUM_HEADS).T.reshape(1, D_MODEL)


@jax.jit
def kernel(query, key, value, query_labels, value_labels,
           Wq, bq, Wk, bk, Wv, bv, Wm, bm):
    xq, xk, xv = query[0], key[0], value[0]
    WqR, WkR, WvR = (_rowperm(Wq).astype(jnp.bfloat16),
                     _rowperm(Wk).astype(jnp.bfloat16),
                     _rowperm(Wv).astype(jnp.bfloat16))
    bqR, bkR, bvR = _rowperm_b(bq), _rowperm_b(bk), _rowperm_b(bv)
    WmP = Wm.reshape(D_MODEL, HEAD_DIM, NUM_HEADS).transpose(0, 2, 1) \
            .reshape(D_MODEL, D_MODEL).astype(jnp.bfloat16)
    bmR = bm.reshape(D_MODEL, 1)

    qlab = query_labels[0].reshape(N, 1)
    vlab_row = value_labels[0].reshape(1, N)
    NAUG = NUM_HEADS * AUG

    nblk = N // QBLK
    q2, qn, k2, kn, v_aug = pl.pallas_call(
        _qkv_kernel,
        grid=(nblk,),
        in_specs=[
            pl.BlockSpec((D_MODEL, QBLK), lambda i: (0, i)),
            pl.BlockSpec((D_MODEL, QBLK), lambda i: (0, i)),
            pl.BlockSpec((D_MODEL, QBLK), lambda i: (0, i)),
            pl.BlockSpec((D_MODEL, D_MODEL), lambda i: (0, 0)),
            pl.BlockSpec((D_MODEL, D_MODEL), lambda i: (0, 0)),
            pl.BlockSpec((D_MODEL, D_MODEL), lambda i: (0, 0)),
            pl.BlockSpec((1, D_MODEL), lambda i: (0, 0)),
            pl.BlockSpec((1, D_MODEL), lambda i: (0, 0)),
            pl.BlockSpec((1, D_MODEL), lambda i: (0, 0)),
        ],
        out_specs=[
            pl.BlockSpec((QBLK, D_MODEL), lambda i: (i, 0)),
            pl.BlockSpec((QBLK, NUM_HEADS), lambda i: (i, 0)),
            pl.BlockSpec((QBLK, D_MODEL), lambda i: (i, 0)),
            pl.BlockSpec((QBLK, NUM_HEADS), lambda i: (i, 0)),
            pl.BlockSpec((QBLK, NAUG), lambda i: (i, 0)),
        ],
        out_shape=[
            jax.ShapeDtypeStruct((N, D_MODEL), jnp.bfloat16),
            jax.ShapeDtypeStruct((N, NUM_HEADS), jnp.float32),
            jax.ShapeDtypeStruct((N, D_MODEL), jnp.bfloat16),
            jax.ShapeDtypeStruct((N, NUM_HEADS), jnp.float32),
            jax.ShapeDtypeStruct((N, NAUG), jnp.bfloat16),
        ],
    )(xq, xk, xv, WqR, WkR, WvR, bqR, bkR, bvR)

    out = pl.pallas_call(
        _attn_kernel,
        grid=(nblk,),
        in_specs=[
            pl.BlockSpec((QBLK, 1), lambda i: (i, 0)),
            pl.BlockSpec((1, N), lambda i: (0, 0)),
            pl.BlockSpec((QBLK, D_MODEL), lambda i: (i, 0)),
            pl.BlockSpec((QBLK, NUM_HEADS), lambda i: (i, 0)),
            pl.BlockSpec((N, D_MODEL), lambda i: (0, 0)),
            pl.BlockSpec((N, NUM_HEADS), lambda i: (0, 0)),
            pl.BlockSpec((N, NAUG), lambda i: (0, 0)),
            pl.BlockSpec((D_MODEL, D_MODEL), lambda i: (0, 0)),
            pl.BlockSpec((D_MODEL, 1), lambda i: (0, 0)),
        ],
        out_specs=pl.BlockSpec((D_MODEL, QBLK), lambda i: (0, i)),
        out_shape=jax.ShapeDtypeStruct((D_MODEL, N), jnp.float32),
    )(qlab, vlab_row, q2, qn, k2, kn, v_aug, WmP, bmR)

    return out[None]
